# Initial kernel scaffold; baseline (speedup 1.0000x reference)
import jax
import jax.numpy as jnp
from jax.experimental import pallas as pl


def kernel(x, edge_index, W0, b0, g0, be0, W1, b1, g1, be1, W2, b2, g2, be2):
    raise NotImplementedError("write your pallas kernel here")


# speedup vs baseline: 6.6815x; 6.6815x over previous
import functools
import math

import jax
import jax.numpy as jnp
from jax import lax
from jax.experimental import pallas as pl
from jax.experimental.pallas import tpu as pltpu
from jax.experimental.pallas import tpu_sc as plsc

_NC = 2
_NS = 16
_NW = _NC * _NS
_BLK = 128


def _sc_degree_body(np_pad, k_blocks, dst_hbm, out_hbm,
                    dst_v, ones_v, zbuf, deg_sh):
    c = lax.axis_index("c")
    s = lax.axis_index("s")
    tile = c * _NS + s
    rows_per_sub = np_pad // _NS

    for i in range(_BLK // 16):
        ones_v[pl.ds(16 * i, 16)] = jnp.full((16,), 1.0, jnp.float32)

    def zero_body(i, _):
        zbuf[pl.ds(16 * i, 16)] = jnp.zeros((16,), jnp.float32)
        return ()

    lax.fori_loop(0, rows_per_sub // 16, zero_body, ())
    pltpu.sync_copy(zbuf, deg_sh.at[pl.ds(s * rows_per_sub, rows_per_sub)])
    plsc.subcore_barrier()

    pltpu.sync_copy(dst_hbm.at[pl.ds(tile * k_blocks, k_blocks)], dst_v)

    def body(j, _):
        pltpu.sync_copy(ones_v, deg_sh.at[dst_v.at[j]], add=True)
        return ()

    lax.fori_loop(0, k_blocks, body, ())
    plsc.subcore_barrier()

    pltpu.sync_copy(deg_sh.at[pl.ds(s * rows_per_sub, rows_per_sub)], zbuf)
    pltpu.sync_copy(zbuf, out_hbm.at[pl.ds(c * np_pad + s * rows_per_sub,
                                           rows_per_sub)])


def _sc_propagate_body(np_pad, k_blocks, h_hbm, src_hbm, dst_hbm,
                       out_hbm, src_v, dst_v, rows_v, acc_sh, sem):
    c = lax.axis_index("c")
    s = lax.axis_index("s")
    tile = c * _NS + s
    rows_per_sub = np_pad // _NS
    hdim = rows_v.shape[1]
    nchunks = rows_per_sub // _BLK

    def zero_body(r, _):
        for i in range(hdim // 16):
            rows_v[r, pl.ds(16 * i, 16)] = jnp.zeros((16,), jnp.float32)
        return ()

    lax.fori_loop(0, _BLK, zero_body, ())
    for t in range(nchunks):
        pltpu.sync_copy(
            rows_v, acc_sh.at[pl.ds(s * rows_per_sub + t * _BLK, _BLK)])
    plsc.subcore_barrier()

    pltpu.sync_copy(src_hbm.at[pl.ds(tile * k_blocks, k_blocks)], src_v)
    pltpu.sync_copy(dst_hbm.at[pl.ds(tile * k_blocks, k_blocks)], dst_v)

    def body(j, _):
        pltpu.async_copy(h_hbm.at[src_v.at[j]], rows_v, sem).wait()
        pltpu.sync_copy(rows_v, acc_sh.at[dst_v.at[j]], add=True)
        return ()

    lax.fori_loop(0, k_blocks, body, ())
    plsc.subcore_barrier()

    for t in range(nchunks):
        base = s * rows_per_sub + t * _BLK
        pltpu.sync_copy(acc_sh.at[pl.ds(base, _BLK)], rows_v)
        pltpu.sync_copy(rows_v, out_hbm.at[c, pl.ds(base, _BLK)])


def _make_sc_degree(np_pad, k_blocks):
    mesh = plsc.VectorSubcoreMesh(core_axis_name="c", subcore_axis_name="s")
    return pl.kernel(
        functools.partial(_sc_degree_body, np_pad, k_blocks),
        out_type=jax.ShapeDtypeStruct((_NC * np_pad,), jnp.float32),
        mesh=mesh,
        scratch_types=[
            pltpu.VMEM((k_blocks, _BLK), jnp.int32),
            pltpu.VMEM((_BLK,), jnp.float32),
            pltpu.VMEM((np_pad // _NS,), jnp.float32),
            pltpu.VMEM_SHARED((np_pad,), jnp.float32),
        ],
    )


def _make_sc_propagate(np_pad, k_blocks, hdim):
    mesh = plsc.VectorSubcoreMesh(core_axis_name="c", subcore_axis_name="s")
    return pl.kernel(
        functools.partial(_sc_propagate_body, np_pad, k_blocks),
        out_type=jax.ShapeDtypeStruct((_NC, np_pad, hdim), jnp.float32),
        mesh=mesh,
        scratch_types=[
            pltpu.VMEM((k_blocks, _BLK), jnp.int32),
            pltpu.VMEM((k_blocks, _BLK), jnp.int32),
            pltpu.VMEM((_BLK, hdim), jnp.float32),
            pltpu.VMEM_SHARED((np_pad, hdim), jnp.float32),
            pltpu.SemaphoreType.DMA,
        ],
    )


def _tc_matmul_body(x_ref, w_ref, o_ref):
    o_ref[...] = jnp.dot(x_ref[...], w_ref[...],
                         preferred_element_type=jnp.float32)


def _tc_mkh0_body(n, h, h_raw_ref, degp_ref, dinv_ref, hp_ref):
    deg = degp_ref[0, pl.ds(0, n)] + degp_ref[1, pl.ds(0, n)] + 1.0
    dinv = lax.rsqrt(deg)
    dinv_ref[...] = dinv[:, None]
    hp_ref[...] = jnp.zeros(hp_ref.shape, jnp.float32)
    hp_ref[pl.ds(0, n), pl.ds(0, h)] = h_raw_ref[...] * dinv[:, None]


def _tc_post_body(n, h, relu_next, accp_ref, hp_ref, dinv_ref, b_ref,
                  g_ref, be_ref, w_ref, out_ref):
    dinv = dinv_ref[...]
    y = dinv * (accp_ref[0, pl.ds(0, n), pl.ds(0, h)]
                + accp_ref[1, pl.ds(0, n), pl.ds(0, h)]
                + hp_ref[pl.ds(0, n), pl.ds(0, h)]) + b_ref[...]
    m = jnp.mean(y, axis=0, keepdims=True)
    v = jnp.mean((y - m) * (y - m), axis=0, keepdims=True)
    yn = (y - m) * lax.rsqrt(v + 1e-5) * g_ref[...] + be_ref[...]
    if relu_next:
        z = jnp.maximum(yn, 0.0)
        hn = jnp.dot(z, w_ref[...], preferred_element_type=jnp.float32) * dinv
        out_ref[...] = jnp.zeros(out_ref.shape, jnp.float32)
        out_ref[pl.ds(0, n), pl.ds(0, h)] = hn
    else:
        out_ref[...] = yn


def kernel(x, edge_index, W0, b0, g0, be0, W1, b1, g1, be1, W2, b2, g2, be2):
    n, f = x.shape
    e = edge_index.shape[1]
    h = W0.shape[1]

    np_pad = ((n + 1 + 255) // 256) * 256
    per_tile = ((e + _NW - 1) // _NW + 8 * _BLK - 1) // (8 * _BLK) * (8 * _BLK)
    k_blocks = per_tile // _BLK
    e_pad = per_tile * _NW

    src = jnp.concatenate(
        [edge_index[0], jnp.full((e_pad - e,), n, jnp.int32)]).reshape(
            _NW * k_blocks, _BLK)
    dst = jnp.concatenate(
        [edge_index[1], jnp.full((e_pad - e,), n, jnp.int32)]).reshape(
            _NW * k_blocks, _BLK)

    degp = _make_sc_degree(np_pad, k_blocks)(dst)
    degp = degp.reshape(_NC, np_pad)

    h0_raw = pl.pallas_call(
        _tc_matmul_body,
        out_shape=jax.ShapeDtypeStruct((n, h), jnp.float32),
    )(x, W0)

    wpad = 128
    dinv, hp = pl.pallas_call(
        functools.partial(_tc_mkh0_body, n, h),
        out_shape=(jax.ShapeDtypeStruct((n, 1), jnp.float32),
                   jax.ShapeDtypeStruct((np_pad, wpad), jnp.float32)),
    )(h0_raw, degp)

    sc_prop = _make_sc_propagate(np_pad, k_blocks, wpad)
    layers = [(b0, g0, be0, W1), (b1, g1, be1, W2), (b2, g2, be2, None)]
    out = None
    for i, (b, g, be, w_next) in enumerate(layers):
        accp = sc_prop(hp, src, dst)
        last = w_next is None
        w_arg = jnp.zeros((h, h), jnp.float32) if last else w_next
        out_shape = jax.ShapeDtypeStruct(
            (n, h) if last else (np_pad, wpad), jnp.float32)
        res = pl.pallas_call(
            functools.partial(_tc_post_body, n, h, not last),
            out_shape=out_shape,
        )(accp, hp, dinv, b.reshape(1, h), g.reshape(1, h),
          be.reshape(1, h), w_arg)
        if last:
            out = res
        else:
            hp = res
    return out


# speedup vs baseline: 7.4613x; 1.1167x over previous
import functools
import math

import jax
import jax.numpy as jnp
from jax import lax
from jax.experimental import pallas as pl
from jax.experimental.pallas import tpu as pltpu
from jax.experimental.pallas import tpu_sc as plsc

_NC = 2
_NS = 16
_NW = _NC * _NS
_BLK = 128
_CHK = 16


def _sc_degree_body(np_pad, k_blocks, dst_hbm, out_hbm,
                    dst_v, ones_v, zbuf, deg_sh):
    c = lax.axis_index("c")
    s = lax.axis_index("s")
    tile = c * _NS + s
    rows_per_sub = np_pad // _NS

    for i in range(_BLK // 16):
        ones_v[pl.ds(16 * i, 16)] = jnp.full((16,), 1.0, jnp.float32)

    def zero_body(i, _):
        zbuf[pl.ds(16 * i, 16)] = jnp.zeros((16,), jnp.float32)
        return ()

    lax.fori_loop(0, rows_per_sub // 16, zero_body, ())
    pltpu.sync_copy(zbuf, deg_sh.at[pl.ds(s * rows_per_sub, rows_per_sub)])
    plsc.subcore_barrier()

    pltpu.sync_copy(dst_hbm.at[pl.ds(tile * k_blocks, k_blocks)], dst_v)

    def body(j, _):
        pltpu.sync_copy(ones_v, deg_sh.at[dst_v.at[j]], add=True)
        return ()

    lax.fori_loop(0, k_blocks, body, ())
    plsc.subcore_barrier()

    pltpu.sync_copy(deg_sh.at[pl.ds(s * rows_per_sub, rows_per_sub)], zbuf)
    pltpu.sync_copy(zbuf, out_hbm.at[pl.ds(c * np_pad + s * rows_per_sub,
                                           rows_per_sub)])


def _sc_propagate_body(np_pad, k_blocks, h_hbm, src_hbm, dst_hbm,
                       out_hbm, src_v0, src_v1, dst_v0, dst_v1, rows_g0,
                       rows_g1, acc_sh, gsem0, gsem1):
    c = lax.axis_index("c")
    s = lax.axis_index("s")
    tile = c * _NS + s
    rows_per_sub = np_pad // _NS
    hdim = rows_g0.shape[1]
    nchunks = rows_per_sub // _BLK
    rows_g = (rows_g0, rows_g1)
    gsem = (gsem0, gsem1)

    def zero_body(r, _):
        for i in range(hdim // 16):
            rows_g0[r, pl.ds(16 * i, 16)] = jnp.zeros((16,), jnp.float32)
        return ()

    lax.fori_loop(0, _BLK, zero_body, ())
    for t in range(nchunks):
        pltpu.sync_copy(
            rows_g0, acc_sh.at[pl.ds(s * rows_per_sub + t * _BLK, _BLK)])
    plsc.subcore_barrier()

    src_c = (src_v0, src_v1)
    dst_c = (dst_v0, dst_v1)

    def load_chunk(q):
        pc = q % 2
        base = tile * k_blocks + q * _CHK
        pltpu.sync_copy(src_hbm.at[pl.ds(base, _CHK)], src_c[pc])
        pltpu.sync_copy(dst_hbm.at[pl.ds(base, _CHK)], dst_c[pc])

    def fire_gather(j, b):
        idx = src_c[(j // _CHK) % 2].at[j % _CHK]
        pltpu.async_copy(h_hbm.at[idx], rows_g[b], gsem[b])

    def wait_gather(b):
        pltpu.make_async_copy(h_hbm.at[src_c[0].at[0]], rows_g[b],
                              gsem[b]).wait()

    load_chunk(0)
    fire_gather(0, 0)
    fire_gather(1, 1)
    for j in range(k_blocks):
        b = j % 2
        wait_gather(b)
        idx = dst_c[(j // _CHK) % 2].at[j % _CHK]
        pltpu.sync_copy(rows_g[b], acc_sh.at[idx], add=True)
        if j + 2 < k_blocks:
            if (j + 2) % _CHK == 0:
                load_chunk((j + 2) // _CHK)
            fire_gather(j + 2, b)
    plsc.subcore_barrier()

    for t in range(nchunks):
        base = s * rows_per_sub + t * _BLK
        pltpu.sync_copy(acc_sh.at[pl.ds(base, _BLK)], rows_g0)
        pltpu.sync_copy(rows_g0, out_hbm.at[c, pl.ds(base, _BLK)])


def _make_sc_degree(np_pad, k_blocks):
    mesh = plsc.VectorSubcoreMesh(core_axis_name="c", subcore_axis_name="s")
    return pl.kernel(
        functools.partial(_sc_degree_body, np_pad, k_blocks),
        out_type=jax.ShapeDtypeStruct((_NC * np_pad,), jnp.float32),
        mesh=mesh,
        scratch_types=[
            pltpu.VMEM((k_blocks, _BLK), jnp.int32),
            pltpu.VMEM((_BLK,), jnp.float32),
            pltpu.VMEM((np_pad // _NS,), jnp.float32),
            pltpu.VMEM_SHARED((np_pad,), jnp.float32),
        ],
    )


def _make_sc_propagate(np_pad, k_blocks, gdim):
    mesh = plsc.VectorSubcoreMesh(core_axis_name="c", subcore_axis_name="s")
    return pl.kernel(
        functools.partial(_sc_propagate_body, np_pad, k_blocks),
        out_type=jax.ShapeDtypeStruct((_NC, np_pad, gdim), jnp.float32),
        mesh=mesh,
        scratch_types=[
            pltpu.VMEM((_CHK, _BLK), jnp.int32),
            pltpu.VMEM((_CHK, _BLK), jnp.int32),
            pltpu.VMEM((_CHK, _BLK), jnp.int32),
            pltpu.VMEM((_CHK, _BLK), jnp.int32),
            pltpu.VMEM((_BLK, gdim), jnp.float32),
            pltpu.VMEM((_BLK, gdim), jnp.float32),
            pltpu.VMEM_SHARED((np_pad, gdim), jnp.float32),
            pltpu.SemaphoreType.DMA,
            pltpu.SemaphoreType.DMA,
        ],
    )


def _tc_matmul_body(x_ref, w_ref, o_ref):
    o_ref[...] = jnp.dot(x_ref[...], w_ref[...],
                         preferred_element_type=jnp.float32)


def _tc_mkh0_body(n, h, h_raw_ref, degp_ref, dinv_ref, hp_ref):
    deg = degp_ref[0, pl.ds(0, n)] + degp_ref[1, pl.ds(0, n)] + 1.0
    dinv = lax.rsqrt(deg)
    dinv_ref[...] = dinv[:, None]
    hp_ref[...] = jnp.zeros(hp_ref.shape, jnp.float32)
    hp_ref[pl.ds(0, n), pl.ds(0, h)] = h_raw_ref[...] * dinv[:, None]


def _tc_post_body(n, h, relu_next, accp_ref, hp_ref, dinv_ref, b_ref,
                  g_ref, be_ref, w_ref, out_ref):
    dinv = dinv_ref[...]
    y = dinv * (accp_ref[0, pl.ds(0, n), pl.ds(0, h)]
                + accp_ref[1, pl.ds(0, n), pl.ds(0, h)]
                + hp_ref[pl.ds(0, n), pl.ds(0, h)]) + b_ref[...]
    m = jnp.mean(y, axis=0, keepdims=True)
    v = jnp.mean((y - m) * (y - m), axis=0, keepdims=True)
    yn = (y - m) * lax.rsqrt(v + 1e-5) * g_ref[...] + be_ref[...]
    if relu_next:
        z = jnp.maximum(yn, 0.0)
        hn = jnp.dot(z, w_ref[...], preferred_element_type=jnp.float32) * dinv
        out_ref[...] = jnp.zeros(out_ref.shape, jnp.float32)
        out_ref[pl.ds(0, n), pl.ds(0, h)] = hn
    else:
        out_ref[...] = yn


def kernel(x, edge_index, W0, b0, g0, be0, W1, b1, g1, be1, W2, b2, g2, be2):
    n, f = x.shape
    e = edge_index.shape[1]
    h = W0.shape[1]

    np_pad = ((n + 1 + 255) // 256) * 256
    per_tile = ((e + _NW - 1) // _NW + 8 * _BLK - 1) // (8 * _BLK) * (8 * _BLK)
    k_blocks = per_tile // _BLK
    e_pad = per_tile * _NW

    src = jnp.concatenate(
        [edge_index[0], jnp.full((e_pad - e,), n, jnp.int32)]).reshape(
            _NW * k_blocks, _BLK)
    dst = jnp.concatenate(
        [edge_index[1], jnp.full((e_pad - e,), n, jnp.int32)]).reshape(
            _NW * k_blocks, _BLK)

    degp = _make_sc_degree(np_pad, k_blocks)(dst)
    degp = degp.reshape(_NC, np_pad)

    h0_raw = pl.pallas_call(
        _tc_matmul_body,
        out_shape=jax.ShapeDtypeStruct((n, h), jnp.float32),
    )(x, W0)

    wpad = 128
    dinv, hp = pl.pallas_call(
        functools.partial(_tc_mkh0_body, n, h),
        out_shape=(jax.ShapeDtypeStruct((n, 1), jnp.float32),
                   jax.ShapeDtypeStruct((np_pad, wpad), jnp.float32)),
    )(h0_raw, degp)

    sc_prop = _make_sc_propagate(np_pad, k_blocks, wpad)
    layers = [(b0, g0, be0, W1), (b1, g1, be1, W2), (b2, g2, be2, None)]
    out = None
    for i, (b, g, be, w_next) in enumerate(layers):
        accp = sc_prop(hp, src, dst)
        last = w_next is None
        w_arg = jnp.zeros((h, h), jnp.float32) if last else w_next
        out_shape = jax.ShapeDtypeStruct(
            (n, h) if last else (np_pad, wpad), jnp.float32)
        res = pl.pallas_call(
            functools.partial(_tc_post_body, n, h, not last),
            out_shape=out_shape,
        )(accp, hp, dinv, b.reshape(1, h), g.reshape(1, h),
          be.reshape(1, h), w_arg)
        if last:
            out = res
        else:
            hp = res
    return out
